# baseline (speedup 1.0000x reference)
import functools

import jax
import jax.numpy as jnp
from jax import lax
from jax.experimental import pallas as pl
from jax.experimental.pallas import tpu as pltpu
from jax.experimental.pallas import tpu_sc as plsc


@functools.lru_cache(maxsize=None)
def _make_sc_gather(V, D, B):
    info = plsc.get_sparse_core_info()
    nw = info.num_cores * info.num_subcores
    b_per_w = B // nw
    assert B % nw == 0 and b_per_w % 8 == 0

    mesh = plsc.VectorSubcoreMesh(core_axis_name="c", subcore_axis_name="s")

    @functools.partial(
        pl.kernel,
        mesh=mesh,
        out_type=jax.ShapeDtypeStruct((B, D), jnp.float32),
        scratch_types=[
            pltpu.VMEM((b_per_w,), jnp.int32),
            pltpu.VMEM((b_per_w, D), jnp.float32),
            pltpu.SemaphoreType.DMA,
        ],
        compiler_params=pltpu.CompilerParams(use_tc_tiling_on_sc=False),
    )
    def gather(idx_hbm, table_hbm, out_hbm, idx_v, rows_v, sem):
        wid = lax.axis_index("s") * info.num_cores + lax.axis_index("c")
        base = wid * b_per_w
        pltpu.sync_copy(idx_hbm.at[pl.ds(base, b_per_w)], idx_v)
        pltpu.async_copy(table_hbm.at[idx_v], rows_v, sem).wait()
        pltpu.sync_copy(rows_v, out_hbm.at[pl.ds(base, b_per_w)])

    return gather


def _make_proj_body(V, v_blk, nbuf, nsteps):
    rem = V - (nsteps - 1) * v_blk
    rem_a = (rem // 128) * 128
    rem_b = rem - rem_a
    last_slot = (nsteps - 1) % nbuf

    def body(emb_ref, w_ref, b_ref, out_hbm, acc_ref, tail_ref, sems, tail_sem):
        j = pl.program_id(0)
        acc = lax.dot_general(
            emb_ref[...], w_ref[...],
            dimension_numbers=(((1,), (1,)), ((), ())),
            preferred_element_type=jnp.float32,
        ) + b_ref[0]

        def full_copy(k):
            return pltpu.make_async_copy(
                acc_ref.at[k],
                out_hbm.at[:, pl.ds(j * v_blk, v_blk)],
                sems.at[k],
            )

        def drain_desc(k):
            return pltpu.make_async_copy(
                acc_ref.at[k],
                out_hbm.at[:, pl.ds(0, v_blk)],
                sems.at[k],
            )

        def tail_copy_a(k):
            return pltpu.make_async_copy(
                acc_ref.at[k, :, pl.ds(0, rem_a)],
                out_hbm.at[:, pl.ds((nsteps - 1) * v_blk, rem_a)],
                sems.at[k],
            )

        def tail_copy_b():
            return pltpu.make_async_copy(
                tail_ref,
                out_hbm.at[:, pl.ds((nsteps - 1) * v_blk + rem_a, rem_b)],
                tail_sem,
            )

        slot = lax.rem(j, nbuf)
        for k in range(nbuf):
            @pl.when(slot == k)
            def _(k=k):
                @pl.when(j >= nbuf)
                def _():
                    drain_desc(k).wait()
                acc_ref[k] = acc
                @pl.when(j < nsteps - 1)
                def _():
                    full_copy(k).start()

        @pl.when(j == nsteps - 1)
        def _():
            tail_copy_a(last_slot).start()
            if rem_b:
                tail_ref[...] = acc[:, rem_a:rem]
                tail_copy_b().start()
            for k in range(nbuf):
                if k == last_slot:
                    tail_copy_a(k).wait()
                else:
                    drain_desc(k).wait()
            if rem_b:
                tail_copy_b().wait()

    return body


@functools.lru_cache(maxsize=None)
def _make_tc_proj(V, D, B, v_blk, nbuf):
    nsteps = pl.cdiv(V, v_blk)
    return pl.pallas_call(
        _make_proj_body(V, v_blk, nbuf, nsteps),
        grid=(nsteps,),
        in_specs=[
            pl.BlockSpec((B, D), lambda j: (0, 0)),
            pl.BlockSpec((v_blk, D), lambda j: (j, 0)),
            pl.BlockSpec((1, 1, v_blk), lambda j: (j, 0, 0)),
        ],
        out_specs=pl.BlockSpec(memory_space=pl.ANY),
        out_shape=jax.ShapeDtypeStruct((B, V), jnp.float32),
        scratch_shapes=[
            pltpu.VMEM((nbuf, B, v_blk), jnp.float32),
            pltpu.VMEM((B, (V % 128) or 128), jnp.float32),
            pltpu.SemaphoreType.DMA((nbuf,)),
            pltpu.SemaphoreType.DMA,
        ],
        compiler_params=pltpu.CompilerParams(
            dimension_semantics=("arbitrary",),
        ),
    )


def kernel(input, table, W, b):
    B = input.shape[0]
    V, D = table.shape
    emb = _make_sc_gather(V, D, B)(input.astype(jnp.int32), table)
    v_blk, nbuf = 2048, 4
    nsteps = -(-V // v_blk)
    b_pad = jnp.pad(b, (0, nsteps * v_blk - V)).reshape(nsteps, 1, v_blk)
    out = _make_tc_proj(V, D, B, v_blk, nbuf)(emb, W, b_pad)
    return out


# speedup vs baseline: 2.6775x; 2.6775x over previous
import functools

import jax
import jax.numpy as jnp
from jax import lax
from jax.experimental import pallas as pl
from jax.experimental.pallas import tpu as pltpu
from jax.experimental.pallas import tpu_sc as plsc


@functools.lru_cache(maxsize=None)
def _make_sc_gather(V, D, B):
    info = plsc.get_sparse_core_info()
    L = info.num_lanes
    nw = info.num_cores * info.num_subcores
    b_per_w = B // nw
    n_el = b_per_w * D
    assert B % nw == 0 and b_per_w % L == 0 and b_per_w % 8 == 0

    mesh = plsc.VectorSubcoreMesh(core_axis_name="c", subcore_axis_name="s")

    @functools.partial(
        pl.kernel,
        mesh=mesh,
        out_type=jax.ShapeDtypeStruct((D, nw, b_per_w), jnp.float32),
        scratch_types=[
            pltpu.VMEM((b_per_w,), jnp.int32),
            pltpu.VMEM((n_el,), jnp.int32),
            pltpu.VMEM((D, b_per_w), jnp.float32),
            pltpu.SemaphoreType.DMA,
        ],
        compiler_params=pltpu.CompilerParams(use_tc_tiling_on_sc=False),
    )
    def gather(idx_hbm, tflat_hbm, out_hbm, idx_v, a_v, rows_v, sem):
        wid = lax.axis_index("s") * info.num_cores + lax.axis_index("c")
        base = wid * b_per_w
        pltpu.sync_copy(idx_hbm.at[pl.ds(base, b_per_w)], idx_v)

        nh = b_per_w // L
        for k in range(n_el // L):
            d, h = divmod(k, nh)
            a_v[pl.ds(k * L, L)] = idx_v[pl.ds(h * L, L)] + (d * V)

        cps = [
            pltpu.make_async_copy(
                tflat_hbm.at[a_v.at[pl.ds(d * b_per_w, b_per_w)]],
                rows_v.at[d],
                sem,
            )
            for d in range(D)
        ]
        for cp in cps:
            cp.start()
        for cp in cps:
            cp.wait()
        pltpu.sync_copy(rows_v, out_hbm.at[:, wid])

    return gather


def _make_proj_body(V, v_blk, nbuf, nsteps):
    rem = V - (nsteps - 1) * v_blk

    def body(ea_ref, eb_ref, wt_ref, b_ref, out_hbm, acc_ref, sems):
        j = pl.program_id(0)
        w = wt_ref[...]
        h = ea_ref.shape[0]
        acc = lax.dot_general(
            w[:h], ea_ref[...],
            dimension_numbers=(((0,), (0,)), ((), ())),
            preferred_element_type=jnp.float32,
        ) + lax.dot_general(
            w[h:], eb_ref[...],
            dimension_numbers=(((0,), (0,)), ((), ())),
            preferred_element_type=jnp.float32,
        ) + jnp.transpose(b_ref[...])

        def full_copy(k):
            return pltpu.make_async_copy(
                acc_ref.at[k],
                out_hbm.at[pl.ds(j * v_blk, v_blk), :],
                sems.at[k],
            )

        def drain_desc(k):
            return pltpu.make_async_copy(
                acc_ref.at[k],
                out_hbm.at[pl.ds(0, v_blk), :],
                sems.at[k],
            )

        def tail_copy(k):
            return pltpu.make_async_copy(
                acc_ref.at[k, pl.ds(0, rem), :],
                out_hbm.at[pl.ds((nsteps - 1) * v_blk, rem), :],
                sems.at[k],
            )

        slot = lax.rem(j, nbuf)
        for k in range(nbuf):
            @pl.when(slot == k)
            def _(k=k):
                @pl.when(j >= nbuf)
                def _():
                    drain_desc(k).wait()
                acc_ref[k] = acc
                @pl.when(j < nsteps - 1)
                def _():
                    full_copy(k).start()

        @pl.when(j == nsteps - 1)
        def _():
            last_slot = (nsteps - 1) % nbuf
            tail_copy(last_slot).start()
            for k in range(nbuf):
                if k == last_slot:
                    tail_copy(k).wait()
                else:
                    drain_desc(k).wait()

    return body


@functools.lru_cache(maxsize=None)
def _make_tc_proj(V, D, B, v_blk, nbuf):
    nsteps = pl.cdiv(V, v_blk)
    return pl.pallas_call(
        _make_proj_body(V, v_blk, nbuf, nsteps),
        grid=(nsteps,),
        in_specs=[
            pl.BlockSpec((D // 2, B), lambda j: (0, 0)),
            pl.BlockSpec((D // 2, B), lambda j: (0, 0)),
            pl.BlockSpec((D, v_blk), lambda j: (0, j)),
            pl.BlockSpec((1, v_blk), lambda j: (0, j)),
        ],
        out_specs=pl.BlockSpec(memory_space=pl.ANY),
        out_shape=jax.ShapeDtypeStruct((V, B), jnp.float32),
        scratch_shapes=[
            pltpu.VMEM((nbuf, v_blk, B), jnp.float32),
            pltpu.SemaphoreType.DMA((nbuf,)),
        ],
        compiler_params=pltpu.CompilerParams(
            dimension_semantics=("arbitrary",),
        ),
    )


def kernel(input, table, W, b):
    B = input.shape[0]
    V, D = table.shape
    idx32 = input.astype(jnp.int32)
    th = table.T
    h = D // 2
    gat = _make_sc_gather(V, h, B)
    ea = gat(idx32, jnp.ravel(th[:h])).reshape(h, B)
    eb = gat(idx32, jnp.ravel(th[h:])).reshape(h, B)
    out_t = _make_tc_proj(V, D, B, 2048, 4)(ea, eb, W.T, b.reshape(1, V))
    return out_t.T


# speedup vs baseline: 3.2181x; 1.2019x over previous
import functools

import jax
import jax.numpy as jnp
from jax import lax
from jax.experimental import pallas as pl
from jax.experimental.pallas import tpu as pltpu
from jax.experimental.pallas import tpu_sc as plsc


@functools.lru_cache(maxsize=None)
def _make_sc_gather(V, D, B):
    info = plsc.get_sparse_core_info()
    L = info.num_lanes
    nw = info.num_cores * info.num_subcores
    b_per_w = B // nw
    n_el = b_per_w * D
    assert B % nw == 0 and b_per_w % L == 0 and b_per_w % 8 == 0

    mesh = plsc.VectorSubcoreMesh(core_axis_name="c", subcore_axis_name="s")

    @functools.partial(
        pl.kernel,
        mesh=mesh,
        out_type=jax.ShapeDtypeStruct((D, nw, b_per_w), jnp.float32),
        scratch_types=[
            pltpu.VMEM((b_per_w,), jnp.int32),
            pltpu.VMEM((n_el,), jnp.int32),
            pltpu.VMEM((D, b_per_w), jnp.float32),
            pltpu.SemaphoreType.DMA,
        ],
        compiler_params=pltpu.CompilerParams(use_tc_tiling_on_sc=False),
    )
    def gather(idx_hbm, tflat_hbm, out_hbm, idx_v, a_v, rows_v, sem):
        wid = lax.axis_index("s") * info.num_cores + lax.axis_index("c")
        base = wid * b_per_w
        pltpu.sync_copy(idx_hbm.at[pl.ds(base, b_per_w)], idx_v)

        nh = b_per_w // L
        for k in range(n_el // L):
            d, h = divmod(k, nh)
            a_v[pl.ds(k * L, L)] = idx_v[pl.ds(h * L, L)] + (d * V)

        cps = [
            pltpu.make_async_copy(
                tflat_hbm.at[a_v.at[pl.ds(d * b_per_w, b_per_w)]],
                rows_v.at[d],
                sem,
            )
            for d in range(D)
        ]
        for cp in cps:
            cp.start()
        for cp in cps:
            cp.wait()
        pltpu.sync_copy(rows_v, out_hbm.at[:, wid])

    return gather


def _make_proj_body(V, v_blk, nbuf, nsteps):
    def body(emb_ref, wt_ref, b_ref, out_ref):
        out_ref[...] = lax.dot_general(
            wt_ref[...], emb_ref[...],
            dimension_numbers=(((0,), (0,)), ((), ())),
            preferred_element_type=jnp.float32,
        ) + jnp.transpose(b_ref[...])

    return body


@functools.lru_cache(maxsize=None)
def _make_tc_proj(V, D, B, v_blk, nbuf):
    nsteps = pl.cdiv(V, v_blk)
    return pl.pallas_call(
        _make_proj_body(V, v_blk, nbuf, nsteps),
        grid=(nsteps,),
        in_specs=[
            pl.BlockSpec((D, B), lambda j: (0, 0)),
            pl.BlockSpec((D, v_blk), lambda j: (0, j)),
            pl.BlockSpec((1, v_blk), lambda j: (0, j)),
        ],
        out_specs=pl.BlockSpec((v_blk, B), lambda j: (j, 0),
                               pipeline_mode=pl.Buffered(buffer_count=2)),
        out_shape=jax.ShapeDtypeStruct((V, B), jnp.float32),
        compiler_params=pltpu.CompilerParams(
            dimension_semantics=("arbitrary",),
        ),
    )


def kernel(input, table, W, b):
    B = input.shape[0]
    V, D = table.shape
    tflat = jnp.ravel(table.T)
    emb_t = _make_sc_gather(V, D, B)(input.astype(jnp.int32), tflat)
    emb_t = emb_t.reshape(D, B)
    out_t = _make_tc_proj(V, D, B, 2048, 4)(emb_t, W.T, b.reshape(1, V))
    return out_t.T
